# baseline (speedup 1.0000x reference)
import jax
import jax.numpy as jnp
import numpy as np
from jax.experimental import pallas as pl

_B, _N, _D = 4, 256, 64
_H, _HD = 4, 64


def _gat_body(x_ref, pool_ref, i8_ref, W0, A0, b0, W1, A1, b1,
              W2, A2, b2, Wr, br, out_ref):
    B, N, H, C = _B, _N, _H, _HD
    x = x_ref[...]
    i8 = i8_ref[...]
    for (W, A, b) in ((W0, A0, b0), (W1, A1, b1), (W2, A2, b2)):
        xp = jnp.dot(x, W[...], preferred_element_type=jnp.float32)
        outs = []
        for bi in range(B):
            r0 = bi * N
            xpb = xp[r0:r0 + N, :]
            scores = jax.lax.dot_general(
                A[...], xpb,
                dimension_numbers=(((1,), (1,)), ((), ())),
                preferred_element_type=jnp.float32)
            Dr = scores[0:H, :]
            Sr = scores[H:2 * H, :]
            smax = jnp.max(Sr, axis=-1, keepdims=True)
            zm = Dr + smax
            m = jnp.maximum(zm, 0.2 * zm)
            Pr = jnp.exp(zm - m)
            Rr = jnp.exp(0.2 * zm - m)
            q = jnp.exp(Sr - smax)
            t = jnp.exp(0.2 * (Sr - smax))
            PRc = jax.lax.dot_general(
                jnp.concatenate([Pr, Rr], axis=0), i8,
                dimension_numbers=(((0,), (0,)), ((), ())),
                preferred_element_type=jnp.float32)
            acc = jnp.zeros((N, C), dtype=jnp.float32)
            for h in range(H):
                xpbh = xpb[:, h * C:(h + 1) * C]
                Pc = PRc[:, h:h + 1]
                Rc = PRc[:, H + h:H + h + 1]
                e = jnp.maximum(Pc * q[h:h + 1, :], Rc * t[h:h + 1, :])
                den = jnp.sum(e, axis=-1, keepdims=True)
                num = jnp.dot(e, xpbh, preferred_element_type=jnp.float32)
                acc = acc + num * (1.0 / (den + 1e-16))
            outs.append(acc)
        x = jax.nn.relu(jnp.concatenate(outs, axis=0) * (1.0 / H) + b[...])
    pooled = jnp.dot(pool_ref[...], x, preferred_element_type=jnp.float32)
    out_ref[...] = (jnp.dot(pooled, Wr[...], preferred_element_type=jnp.float32)
                    + br[...])


def _blockdiag_scores(a_s, a_d):
    H, C = a_s.shape
    A = jnp.zeros((2 * H, H * C), dtype=jnp.float32)
    for h in range(H):
        A = A.at[h, h * C:(h + 1) * C].set(a_d[h])
        A = A.at[H + h, h * C:(h + 1) * C].set(a_s[h])
    return A


def kernel(embeddings, W0, as0, ad0, b0, W1, as1, ad1, b1, W2, as2, ad2, b2,
           Wr, br):
    A0 = _blockdiag_scores(as0, ad0)
    A1 = _blockdiag_scores(as1, ad1)
    A2 = _blockdiag_scores(as2, ad2)
    xflat = embeddings.reshape(_B * _N, _D)
    pool = jnp.asarray(
        np.kron(np.eye(_B, dtype=np.float32),
                np.full((1, _N), 1.0 / _N, dtype=np.float32)))
    i8 = jnp.eye(2 * _H, dtype=jnp.float32)
    return pl.pallas_call(
        _gat_body,
        out_shape=jax.ShapeDtypeStruct((_B, _D), jnp.float32),
    )(xflat, pool, i8, W0, A0, b0, W1, A1, b1, W2, A2, b2, Wr, br)


# speedup vs baseline: 1.3255x; 1.3255x over previous
import jax
import jax.numpy as jnp
from jax.experimental import pallas as pl

_B, _N, _D = 4, 256, 64
_H, _HD = 4, 64


def _gat_body(x_ref, W0, as0, ad0, b0, W1, as1, ad1, b1, W2, as2, ad2, b2,
              Wr, br, out_ref):
    B, N, H, C = _B, _N, _H, _HD
    rr = jax.lax.broadcasted_iota(jnp.int32, (2 * H, 2 * H), 0)
    cc = jax.lax.broadcasted_iota(jnp.int32, (2 * H, 2 * H), 1)
    i8 = (rr == cc).astype(jnp.float32)
    rows = []
    for bi in range(B):
        x = x_ref[bi]
        for (W, a_s, a_d, b) in ((W0, as0, ad0, b0),
                                 (W1, as1, ad1, b1),
                                 (W2, as2, ad2, b2)):
            xp = jnp.dot(x, W[...], preferred_element_type=jnp.float32)
            qs, ts, prs = [], [], []
            for h in range(H):
                xpbh = xp[:, h * C:(h + 1) * C]
                dr = jax.lax.dot_general(
                    a_d[h:h + 1, :], xpbh,
                    dimension_numbers=(((1,), (1,)), ((), ())),
                    preferred_element_type=jnp.float32)
                sr = jax.lax.dot_general(
                    a_s[h:h + 1, :], xpbh,
                    dimension_numbers=(((1,), (1,)), ((), ())),
                    preferred_element_type=jnp.float32)
                smax = jnp.max(sr, axis=-1, keepdims=True)
                zm = dr + smax
                m = jnp.maximum(zm, 0.2 * zm)
                prs.append(jnp.exp(zm - m))
                prs.append(jnp.exp(0.2 * zm - m))
                qs.append(jnp.exp(sr - smax))
                ts.append(jnp.exp(0.2 * (sr - smax)))
            PRc = jax.lax.dot_general(
                jnp.concatenate(prs, axis=0), i8,
                dimension_numbers=(((0,), (0,)), ((), ())),
                preferred_element_type=jnp.float32)
            acc = jnp.zeros((N, C), dtype=jnp.float32)
            for h in range(H):
                xpbh = xp[:, h * C:(h + 1) * C]
                Pc = PRc[:, 2 * h:2 * h + 1]
                Rc = PRc[:, 2 * h + 1:2 * h + 2]
                e = jnp.maximum(Pc * qs[h], Rc * ts[h])
                den = jnp.sum(e, axis=-1, keepdims=True)
                num = jnp.dot(e, xpbh, preferred_element_type=jnp.float32)
                acc = acc + num * (1.0 / (den + 1e-16))
            x = jax.nn.relu(acc * (1.0 / H) + b[...])
        pooled = jnp.mean(x, axis=0, keepdims=True)
        rows.append(jnp.dot(pooled, Wr[...],
                            preferred_element_type=jnp.float32) + br[...])
    out_ref[...] = jnp.concatenate(rows, axis=0)


def kernel(embeddings, W0, as0, ad0, b0, W1, as1, ad1, b1, W2, as2, ad2, b2,
           Wr, br):
    return pl.pallas_call(
        _gat_body,
        out_shape=jax.ShapeDtypeStruct((_B, _D), jnp.float32),
    )(embeddings, W0, as0, ad0, b0, W1, as1, ad1, b1, W2, as2, ad2, b2,
      Wr, br)


# speedup vs baseline: 1.5501x; 1.1695x over previous
import jax
import jax.numpy as jnp
from jax.experimental import pallas as pl

_B, _N, _D = 4, 256, 64
_H, _HD = 4, 64


def _gat_body(x_ref, W0, as0, ad0, b0, W1, as1, ad1, b1, W2, as2, ad2, b2,
              Wr, br, out_ref):
    B, N, H, C = _B, _N, _H, _HD
    r64 = jax.lax.broadcasted_iota(jnp.int32, (C, H * C), 0)
    c256 = jax.lax.broadcasted_iota(jnp.int32, (C, H * C), 1)
    E = [(c256 - h * C == r64).astype(jnp.float32) for h in range(H)]
    rr = jax.lax.broadcasted_iota(jnp.int32, (2 * H, 2 * H), 0)
    cc = jax.lax.broadcasted_iota(jnp.int32, (2 * H, 2 * H), 1)
    i8 = (rr == cc).astype(jnp.float32)

    x = jnp.concatenate([x_ref[bi] for bi in range(B)], axis=0)
    for (W, a_s, a_d, b) in ((W0, as0, ad0, b0),
                             (W1, as1, ad1, b1),
                             (W2, as2, ad2, b2)):
        xp = jnp.dot(x, W[...], preferred_element_type=jnp.float32)
        A = jnp.concatenate(
            [jnp.dot(a_d[h:h + 1, :], E[h], preferred_element_type=jnp.float32)
             for h in range(H)] +
            [jnp.dot(a_s[h:h + 1, :], E[h], preferred_element_type=jnp.float32)
             for h in range(H)], axis=0)
        outs = []
        for bi in range(B):
            xpb = xp[bi * N:(bi + 1) * N, :]
            scores = jax.lax.dot_general(
                A, xpb,
                dimension_numbers=(((1,), (1,)), ((), ())),
                preferred_element_type=jnp.float32)
            Dr = scores[0:H, :]
            Sr = scores[H:2 * H, :]
            smax = jnp.max(Sr, axis=-1, keepdims=True)
            zm = Dr + smax
            m = jnp.maximum(zm, 0.2 * zm)
            Pr = jnp.exp(zm - m)
            Rr = jnp.exp(0.2 * zm - m)
            q = jnp.exp(Sr - smax)
            t = jnp.exp(0.2 * (Sr - smax))
            PRc = jax.lax.dot_general(
                jnp.concatenate([Pr, Rr], axis=0), i8,
                dimension_numbers=(((0,), (0,)), ((), ())),
                preferred_element_type=jnp.float32)
            acc = jnp.zeros((N, C), dtype=jnp.float32)
            for h in range(H):
                xpbh = xpb[:, h * C:(h + 1) * C]
                Pc = PRc[:, h:h + 1]
                Rc = PRc[:, H + h:H + h + 1]
                e = jnp.maximum(Pc * q[h:h + 1, :], Rc * t[h:h + 1, :])
                den = jnp.sum(e, axis=-1, keepdims=True)
                num = jnp.dot(e, xpbh, preferred_element_type=jnp.float32)
                acc = acc + num * (1.0 / (den + 1e-16))
            outs.append(acc)
        x = jax.nn.relu(jnp.concatenate(outs, axis=0) * (1.0 / H) + b[...])
    pooled = jnp.concatenate(
        [jnp.mean(x[bi * N:(bi + 1) * N, :], axis=0, keepdims=True)
         for bi in range(B)], axis=0)
    out_ref[...] = (jnp.dot(pooled, Wr[...], preferred_element_type=jnp.float32)
                    + br[...])


def kernel(embeddings, W0, as0, ad0, b0, W1, as1, ad1, b1, W2, as2, ad2, b2,
           Wr, br):
    return pl.pallas_call(
        _gat_body,
        out_shape=jax.ShapeDtypeStruct((_B, _D), jnp.float32),
    )(embeddings, W0, as0, ad0, b0, W1, as1, ad1, b1, W2, as2, ad2, b2,
      Wr, br)


# speedup vs baseline: 1.7999x; 1.1612x over previous
import jax
import jax.numpy as jnp
from jax.experimental import pallas as pl

_B, _N, _D = 4, 256, 64
_H, _HD = 4, 64


def _gat_body(x_ref, W0, as0, ad0, b0, W1, as1, ad1, b1, W2, as2, ad2, b2,
              Wr, br, out_ref):
    B, N, H, C = _B, _N, _H, _HD
    r64 = jax.lax.broadcasted_iota(jnp.int32, (C, H * C), 0)
    c256 = jax.lax.broadcasted_iota(jnp.int32, (C, H * C), 1)
    E = [(c256 - h * C == r64).astype(jnp.float32) for h in range(H)]
    rr = jax.lax.broadcasted_iota(jnp.int32, (2 * H * B, 2 * H * B), 0)
    cc = jax.lax.broadcasted_iota(jnp.int32, (2 * H * B, 2 * H * B), 1)
    i32 = (rr == cc).astype(jnp.float32)

    x = jnp.concatenate([x_ref[bi] for bi in range(B)], axis=0)
    for (W, a_s, a_d, b) in ((W0, as0, ad0, b0),
                             (W1, as1, ad1, b1),
                             (W2, as2, ad2, b2)):
        xp = jnp.dot(x, W[...], preferred_element_type=jnp.float32)
        A = jnp.concatenate(
            [jnp.dot(a_d[h:h + 1, :], E[h], preferred_element_type=jnp.float32)
             for h in range(H)] +
            [jnp.dot(a_s[h:h + 1, :], E[h], preferred_element_type=jnp.float32)
             for h in range(H)], axis=0)
        prs, qts = [], []
        for bi in range(B):
            xpb = xp[bi * N:(bi + 1) * N, :]
            scores = jax.lax.dot_general(
                A, xpb,
                dimension_numbers=(((1,), (1,)), ((), ())),
                preferred_element_type=jnp.float32)
            Dr = scores[0:H, :]
            Sr = scores[H:2 * H, :]
            smax = jnp.max(Sr, axis=-1, keepdims=True)
            zm = Dr + smax
            m = jnp.maximum(zm, 0.2 * zm)
            prs.append(jnp.exp(zm - m))
            prs.append(jnp.exp(0.2 * zm - m))
            qts.append((jnp.exp(Sr - smax), jnp.exp(0.2 * (Sr - smax))))
        PRc = jax.lax.dot_general(
            jnp.concatenate(prs, axis=0), i32,
            dimension_numbers=(((0,), (0,)), ((), ())),
            preferred_element_type=jnp.float32)
        outs = []
        for bi in range(B):
            xpb = xp[bi * N:(bi + 1) * N, :]
            q, t = qts[bi]
            acc = jnp.zeros((N, C), dtype=jnp.float32)
            for h in range(H):
                xpbh = xpb[:, h * C:(h + 1) * C]
                c0 = 2 * H * bi
                Pc = PRc[:, c0 + h:c0 + h + 1]
                Rc = PRc[:, c0 + H + h:c0 + H + h + 1]
                e = jnp.maximum(Pc * q[h:h + 1, :], Rc * t[h:h + 1, :])
                den = jnp.sum(e, axis=-1, keepdims=True)
                num = jnp.dot(e, xpbh, preferred_element_type=jnp.float32)
                acc = acc + num * (1.0 / (den + 1e-16))
            outs.append(acc)
        x = jax.nn.relu(jnp.concatenate(outs, axis=0) * (1.0 / H) + b[...])
    pooled = jnp.concatenate(
        [jnp.mean(x[bi * N:(bi + 1) * N, :], axis=0, keepdims=True)
         for bi in range(B)], axis=0)
    out_ref[...] = (jnp.dot(pooled, Wr[...], preferred_element_type=jnp.float32)
                    + br[...])


def kernel(embeddings, W0, as0, ad0, b0, W1, as1, ad1, b1, W2, as2, ad2, b2,
           Wr, br):
    return pl.pallas_call(
        _gat_body,
        out_shape=jax.ShapeDtypeStruct((_B, _D), jnp.float32),
    )(embeddings, W0, as0, ad0, b0, W1, as1, ad1, b1, W2, as2, ad2, b2,
      Wr, br)
